# baseline (speedup 1.0000x reference)
import functools

import jax
import jax.numpy as jnp
from jax import lax
from jax.experimental import pallas as pl
from jax.experimental.pallas import tpu as pltpu
from jax.experimental.pallas import tpu_sc as plsc

_L = 16
_ACC_ROWS = 8
_CNT_ROW = 5
_SUPER = 16
_ALIGN = 128


def _sc_scan_body(chunk, win, nsuper, n_rel, num_cores, e_total,
                  x_hbm, edge_hbm, typ_hbm, part_hbm,
                  edge_v, typ_v, idx_v, rows_v, acc_v, cnt_v, sem):
    sid = lax.axis_index("s")
    wid = sid * num_cores + lax.axis_index("c")
    start = wid * chunk
    astart = jnp.minimum((start // _ALIGN) * _ALIGN, e_total - win)
    delta = start - astart

    cp_edge = pltpu.async_copy(edge_hbm.at[:, pl.ds(astart, win)],
                               edge_v.at[:, pl.ds(0, win)], sem)
    cp_typ = pltpu.async_copy(typ_hbm.at[pl.ds(start, chunk)],
                              typ_v.at[pl.ds(0, chunk)], sem)

    zf = jnp.zeros((_L,), jnp.float32)
    for row in range(_ACC_ROWS):
        cnt_v[row, :] = zf
        for j in range(rows_v.shape[1] // _L):
            acc_v[row, pl.ds(j * _L, _L)] = zf

    cp_edge.wait()
    cp_typ.wait()

    ones_i = jnp.ones((_L,), jnp.int32)
    lanes = lax.iota(jnp.int32, _L)
    tail = (chunk // _L) * _L
    rem = chunk - tail
    if rem:
        blk = edge_v[1, pl.ds(delta + tail, _L)]
        edge_v[1, pl.ds(delta + tail, _L)] = jnp.where(lanes < rem, blk,
                                                       ones_i)
    for b in range((chunk + _L - 1) // _L, nsuper * _SUPER):
        edge_v[1, pl.ds(delta + b * _L, _L)] = ones_i

    def fine_step(b, carry):
        off = delta + b * _L
        dstv = edge_v[1, pl.ds(off, _L)]
        mask = dstv == 0
        nhit = jnp.sum(mask.astype(jnp.int32))

        @pl.when(nhit > 0)
        def _hit():
            dstv2 = edge_v[1, pl.ds(off, _L)]
            mask2 = dstv2 == 0
            srcv = edge_v[0, pl.ds(off, _L)]
            typv = typ_v[pl.ds(b * _L, _L)]
            idx_v[...] = jnp.where(mask2, srcv, 0)
            tdx = jnp.where(mask2, typv, _ACC_ROWS - 1)
            hit_i = jnp.where(mask2, 1, 0)
            pltpu.sync_copy(x_hbm.at[idx_v], rows_v)
            nchunks = rows_v.shape[1] // _L
            for lane in range(_L):
                @pl.when(hit_i[lane] > 0)
                def _add(lane=lane):
                    t_l = tdx[lane]
                    for j in range(nchunks):
                        sl = pl.ds(j * _L, _L)
                        plsc.addupdate(acc_v.at[t_l, sl], rows_v[lane, sl])
            ones_f = jnp.where(mask2, 1.0, 0.0).astype(jnp.float32)
            zero_f = jnp.zeros((_L,), jnp.float32)
            for r in range(n_rel):
                cnt_v[r, :] = cnt_v[r, :] + jnp.where(typv == r, ones_f,
                                                      zero_f)

        return carry

    def super_step(s, carry):
        soff = delta + s * _SUPER * _L
        mv = edge_v[1, pl.ds(soff, _L)]
        for b in range(1, _SUPER):
            mv = jnp.minimum(mv, edge_v[1, pl.ds(soff + b * _L, _L)])
        nz = jnp.sum(jnp.where(mv == 0, 1, 0))

        @pl.when(nz > 0)
        def _scan_fine():
            lax.fori_loop(s * _SUPER, (s + 1) * _SUPER, fine_step, 0)

        return carry

    lax.fori_loop(0, nsuper, super_step, 0)

    for r in range(n_rel):
        acc_v[_CNT_ROW, pl.ds(r * _L, _L)] = cnt_v[r, :]
    pltpu.sync_copy(acc_v, part_hbm.at[wid])


def _tc_tail_body(n_rel, part_ref, comp_ref, bases_ref, root_ref,
                  bias_ref, x0_ref, wg_ref, bg_ref, ws_ref, bs_ref,
                  outg_ref, outs_ref):
    s = jnp.sum(part_ref[...], axis=0)
    rows = []
    for r in range(n_rel):
        cnt_r = jnp.sum(s[_CNT_ROW, r * _L:(r + 1) * _L])
        rows.append(s[r:r + 1, :] / jnp.maximum(cnt_r, 1.0))
    m = jnp.concatenate(rows, axis=0)
    v = lax.dot_general(comp_ref[...], m,
                        (((0,), (0,)), ((), ())),
                        precision=lax.Precision.HIGHEST,
                        preferred_element_type=jnp.float32)
    out0 = jnp.dot(x0_ref[...], root_ref[...],
                   precision=lax.Precision.HIGHEST,
                   preferred_element_type=jnp.float32) + bias_ref[...]
    for r in range(n_rel):
        out0 = out0 + jnp.dot(v[r:r + 1, :], bases_ref[r],
                              precision=lax.Precision.HIGHEST,
                              preferred_element_type=jnp.float32)
    h = jnp.maximum(out0, 0.0)

    def head(w_ref, b_ref, o_ref):
        lg = lax.dot_general(h, w_ref[...], (((1,), (1,)), ((), ())),
                             precision=lax.Precision.HIGHEST,
                             preferred_element_type=jnp.float32) + b_ref[...]
        mx = jnp.max(lg, axis=1, keepdims=True)
        lse = jnp.log(jnp.sum(jnp.exp(lg - mx), axis=1, keepdims=True))
        o_ref[...] = lg - mx - lse

    head(wg_ref, bg_ref, outg_ref)
    head(ws_ref, bs_ref, outs_ref)


def kernel(batch_x, batch_edge_index, batch_edge_type, comp, bases, root,
           bias, W_global, b_global, W_sense, b_sense):
    n, d = batch_x.shape
    e = batch_edge_type.shape[0]
    n_rel = comp.shape[0]
    g = W_global.shape[0]
    s_cnt = W_sense.shape[0]

    info = plsc.get_sparse_core_info()
    nw = info.num_cores * info.num_subcores
    chunk = e // nw
    nblk = (chunk + _L - 1) // _L
    nsuper = (nblk + _SUPER - 1) // _SUPER
    win = ((chunk + 2 * _ALIGN + _ALIGN - 1) // _ALIGN) * _ALIGN
    buf_minor = (win - chunk) + nsuper * _SUPER * _L

    sc_fn = pl.kernel(
        functools.partial(_sc_scan_body, chunk, win, nsuper, n_rel,
                          info.num_cores, e),
        out_type=jax.ShapeDtypeStruct((nw, _ACC_ROWS, d), jnp.float32),
        mesh=plsc.VectorSubcoreMesh(core_axis_name="c", subcore_axis_name="s"),
        compiler_params=pltpu.CompilerParams(needs_layout_passes=False),
        scratch_types=[
            pltpu.VMEM((2, buf_minor), jnp.int32),
            pltpu.VMEM((nblk * _L,), jnp.int32),
            pltpu.VMEM((_L,), jnp.int32),
            pltpu.VMEM((_L, d), jnp.float32),
            pltpu.VMEM((_ACC_ROWS, d), jnp.float32),
            pltpu.VMEM((_ACC_ROWS, _L), jnp.float32),
            pltpu.SemaphoreType.DMA,
        ],
    )
    part = sc_fn(batch_x, batch_edge_index, batch_edge_type)

    x0 = lax.slice(batch_x, (0, 0), (1, d))
    outg, outs = pl.pallas_call(
        functools.partial(_tc_tail_body, n_rel),
        out_shape=(jax.ShapeDtypeStruct((1, g), jnp.float32),
                   jax.ShapeDtypeStruct((1, s_cnt), jnp.float32)),
    )(part, comp, bases, root, bias.reshape(1, d), x0,
      W_global, b_global.reshape(1, g), W_sense, b_sense.reshape(1, s_cnt))

    return (outg.reshape(g), outs.reshape(s_cnt))


# speedup vs baseline: 1.1005x; 1.1005x over previous
import functools

import jax
import jax.numpy as jnp
from jax import lax
from jax.experimental import pallas as pl
from jax.experimental.pallas import tpu as pltpu
from jax.experimental.pallas import tpu_sc as plsc

_L = 16
_ACC_ROWS = 8
_CNT_ROW = 5
_SUPER = 32
_ALIGN = 128


def _sc_scan_body(chunk, win, nsuper, n_rel, num_cores, e_total,
                  x_hbm, edge_hbm, typ_hbm, part_hbm,
                  edge_v, typ_v, idx_v, rows_v, acc_v, cnt_v, sem):
    sid = lax.axis_index("s")
    wid = sid * num_cores + lax.axis_index("c")
    start = wid * chunk
    astart = jnp.minimum((start // _ALIGN) * _ALIGN, e_total - win)
    delta = start - astart

    cp_edge = pltpu.async_copy(edge_hbm.at[:, pl.ds(astart, win)],
                               edge_v.at[:, pl.ds(0, win)], sem)
    cp_typ = pltpu.async_copy(typ_hbm.at[pl.ds(start, chunk)],
                              typ_v.at[pl.ds(0, chunk)], sem)

    zf = jnp.zeros((_L,), jnp.float32)
    for row in range(_ACC_ROWS):
        cnt_v[row, :] = zf
        for j in range(rows_v.shape[1] // _L):
            acc_v[row, pl.ds(j * _L, _L)] = zf

    cp_edge.wait()
    cp_typ.wait()

    ones_i = jnp.ones((_L,), jnp.int32)
    lanes = lax.iota(jnp.int32, _L)
    tail = (chunk // _L) * _L
    rem = chunk - tail
    if rem:
        blk = edge_v[1, pl.ds(delta + tail, _L)]
        edge_v[1, pl.ds(delta + tail, _L)] = jnp.where(lanes < rem, blk,
                                                       ones_i)
    for b in range((chunk + _L - 1) // _L, nsuper * _SUPER):
        edge_v[1, pl.ds(delta + b * _L, _L)] = ones_i

    def fine_step(b, carry):
        off = delta + b * _L
        dstv = edge_v[1, pl.ds(off, _L)]
        mask = dstv == 0
        nhit = jnp.sum(mask.astype(jnp.int32))

        @pl.when(nhit > 0)
        def _hit():
            dstv2 = edge_v[1, pl.ds(off, _L)]
            mask2 = dstv2 == 0
            srcv = edge_v[0, pl.ds(off, _L)]
            typv = typ_v[pl.ds(b * _L, _L)]
            idx_v[...] = jnp.where(mask2, srcv, 0)
            tdx = jnp.where(mask2, typv, _ACC_ROWS - 1)
            hit_i = jnp.where(mask2, 1, 0)
            cp = pltpu.async_copy(x_hbm.at[idx_v], rows_v, sem)
            ones_f = jnp.where(mask2, 1.0, 0.0).astype(jnp.float32)
            zero_f = jnp.zeros((_L,), jnp.float32)
            for r in range(n_rel):
                cnt_v[r, :] = cnt_v[r, :] + jnp.where(typv == r, ones_f,
                                                      zero_f)
            cp.wait()
            nchunks = rows_v.shape[1] // _L
            for lane in range(_L):
                @pl.when(hit_i[lane] > 0)
                def _add(lane=lane):
                    t_l = tdx[lane]
                    for j in range(nchunks):
                        sl = pl.ds(j * _L, _L)
                        plsc.addupdate(acc_v.at[t_l, sl], rows_v[lane, sl])

        return carry

    def super_step(s, carry):
        soff = delta + s * _SUPER * _L
        mv = edge_v[1, pl.ds(soff, _L)]
        for b in range(1, _SUPER):
            mv = jnp.minimum(mv, edge_v[1, pl.ds(soff + b * _L, _L)])
        nz = jnp.sum(jnp.where(mv == 0, 1, 0))

        @pl.when(nz > 0)
        def _scan_fine():
            lax.fori_loop(s * _SUPER, (s + 1) * _SUPER, fine_step, 0)

        return carry

    lax.fori_loop(0, nsuper, super_step, 0)

    for r in range(n_rel):
        acc_v[_CNT_ROW, pl.ds(r * _L, _L)] = cnt_v[r, :]
    pltpu.sync_copy(acc_v, part_hbm.at[wid])


def _tc_tail_body(n_rel, part_ref, comp_ref, bases_ref, root_ref,
                  bias_ref, x0_ref, wg_ref, bg_ref, ws_ref, bs_ref,
                  outg_ref, outs_ref):
    s = jnp.sum(part_ref[...], axis=0)
    rows = []
    for r in range(n_rel):
        cnt_r = jnp.sum(s[_CNT_ROW, r * _L:(r + 1) * _L])
        rows.append(s[r:r + 1, :] / jnp.maximum(cnt_r, 1.0))
    m = jnp.concatenate(rows, axis=0)
    v = lax.dot_general(comp_ref[...], m,
                        (((0,), (0,)), ((), ())),
                        preferred_element_type=jnp.float32)
    out0 = jnp.dot(x0_ref[...], root_ref[...],
                   preferred_element_type=jnp.float32) + bias_ref[...]
    for r in range(n_rel):
        out0 = out0 + jnp.dot(v[r:r + 1, :], bases_ref[r],
                              preferred_element_type=jnp.float32)
    h = jnp.maximum(out0, 0.0)

    def head(w_ref, b_ref, o_ref):
        lg = lax.dot_general(h, w_ref[...], (((1,), (1,)), ((), ())),
                             preferred_element_type=jnp.float32) + b_ref[...]
        mx = jnp.max(lg, axis=1, keepdims=True)
        lse = jnp.log(jnp.sum(jnp.exp(lg - mx), axis=1, keepdims=True))
        o_ref[...] = lg - mx - lse

    head(wg_ref, bg_ref, outg_ref)
    head(ws_ref, bs_ref, outs_ref)


def kernel(batch_x, batch_edge_index, batch_edge_type, comp, bases, root,
           bias, W_global, b_global, W_sense, b_sense):
    n, d = batch_x.shape
    e = batch_edge_type.shape[0]
    n_rel = comp.shape[0]
    g = W_global.shape[0]
    s_cnt = W_sense.shape[0]

    info = plsc.get_sparse_core_info()
    nw = info.num_cores * info.num_subcores
    chunk = e // nw
    nblk = (chunk + _L - 1) // _L
    nsuper = (nblk + _SUPER - 1) // _SUPER
    win = ((chunk + 2 * _ALIGN + _ALIGN - 1) // _ALIGN) * _ALIGN
    buf_minor = (win - chunk) + nsuper * _SUPER * _L

    sc_fn = pl.kernel(
        functools.partial(_sc_scan_body, chunk, win, nsuper, n_rel,
                          info.num_cores, e),
        out_type=jax.ShapeDtypeStruct((nw, _ACC_ROWS, d), jnp.float32),
        mesh=plsc.VectorSubcoreMesh(core_axis_name="c", subcore_axis_name="s"),
        compiler_params=pltpu.CompilerParams(needs_layout_passes=False),
        scratch_types=[
            pltpu.VMEM((2, buf_minor), jnp.int32),
            pltpu.VMEM((nblk * _L,), jnp.int32),
            pltpu.VMEM((_L,), jnp.int32),
            pltpu.VMEM((_L, d), jnp.float32),
            pltpu.VMEM((_ACC_ROWS, d), jnp.float32),
            pltpu.VMEM((_ACC_ROWS, _L), jnp.float32),
            pltpu.SemaphoreType.DMA,
        ],
    )
    part = sc_fn(batch_x, batch_edge_index, batch_edge_type)

    x0 = lax.slice(batch_x, (0, 0), (1, d))
    outg, outs = pl.pallas_call(
        functools.partial(_tc_tail_body, n_rel),
        out_shape=(jax.ShapeDtypeStruct((1, g), jnp.float32),
                   jax.ShapeDtypeStruct((1, s_cnt), jnp.float32)),
    )(part, comp, bases, root, bias.reshape(1, d), x0,
      W_global, b_global.reshape(1, g), W_sense, b_sense.reshape(1, s_cnt))

    return (outg.reshape(g), outs.reshape(s_cnt))


# speedup vs baseline: 1.1646x; 1.0583x over previous
import functools

import jax
import jax.numpy as jnp
from jax import lax
from jax.experimental import pallas as pl
from jax.experimental.pallas import tpu as pltpu
from jax.experimental.pallas import tpu_sc as plsc

_L = 16
_ACC_ROWS = 8
_CNT_ROW = 5
_SUPER = 32
_ALIGN = 128


def _sc_scan_body(chunk, win, nsuper, n_rel, num_cores, e_total,
                  x_hbm, edge_hbm, typ_hbm, part_hbm,
                  edge_v, typ_v, idx_v, rows_v, acc_v, cnt_v, sem):
    sid = lax.axis_index("s")
    wid = sid * num_cores + lax.axis_index("c")
    start = wid * chunk
    astart = jnp.minimum((start // _ALIGN) * _ALIGN, e_total - win)
    delta = start - astart

    cp_edge = pltpu.async_copy(edge_hbm.at[:, pl.ds(astart, win)],
                               edge_v.at[:, pl.ds(0, win)], sem)
    cp_typ = pltpu.async_copy(typ_hbm.at[pl.ds(start, chunk)],
                              typ_v.at[pl.ds(0, chunk)], sem)

    zf = jnp.zeros((_L,), jnp.float32)
    nchunks_all = _ACC_ROWS * rows_v.shape[1] // _L

    ncol = rows_v.shape[1] // _L

    def zinit(j, carry):
        acc_v[j // ncol, pl.ds((j % ncol) * _L, _L)] = zf
        return carry

    lax.fori_loop(0, nchunks_all, zinit, 0)
    for row in range(_ACC_ROWS):
        cnt_v[row, :] = zf

    cp_edge.wait()
    cp_typ.wait()

    ones_i = jnp.ones((_L,), jnp.int32)
    lanes = lax.iota(jnp.int32, _L)
    tail = (chunk // _L) * _L
    rem = chunk - tail
    if rem:
        blk = edge_v[1, pl.ds(delta + tail, _L)]
        edge_v[1, pl.ds(delta + tail, _L)] = jnp.where(lanes < rem, blk,
                                                       ones_i)
    for b in range((chunk + _L - 1) // _L, nsuper * _SUPER):
        edge_v[1, pl.ds(delta + b * _L, _L)] = ones_i

    def fine_step(b, carry):
        off = delta + b * _L
        dstv = edge_v[1, pl.ds(off, _L)]
        mask = dstv == 0
        nhit = jnp.sum(mask.astype(jnp.int32))

        @pl.when(nhit > 0)
        def _hit():
            dstv2 = edge_v[1, pl.ds(off, _L)]
            mask2 = dstv2 == 0
            srcv = edge_v[0, pl.ds(off, _L)]
            typv = typ_v[pl.ds(b * _L, _L)]
            idx_v[...] = jnp.where(mask2, srcv, 0)
            tdx = jnp.where(mask2, typv, _ACC_ROWS - 1)
            hit_i = jnp.where(mask2, 1, 0)
            cp = pltpu.async_copy(x_hbm.at[idx_v], rows_v, sem)
            ones_f = jnp.where(mask2, 1.0, 0.0).astype(jnp.float32)
            zero_f = jnp.zeros((_L,), jnp.float32)
            for r in range(n_rel):
                cnt_v[r, :] = cnt_v[r, :] + jnp.where(typv == r, ones_f,
                                                      zero_f)
            cp.wait()
            nchunks = rows_v.shape[1] // _L
            for lane in range(_L):
                @pl.when(hit_i[lane] > 0)
                def _add(lane=lane):
                    t_l = tdx[lane]

                    def addj(j, carry):
                        sl = pl.ds(j * _L, _L)
                        plsc.addupdate(acc_v.at[t_l, sl],
                                       rows_v[lane, sl])
                        return carry

                    lax.fori_loop(0, nchunks, addj, 0)

        return carry

    def super_step(s, carry):
        soff = delta + s * _SUPER * _L
        mv = edge_v[1, pl.ds(soff, _L)]
        for b in range(1, _SUPER):
            mv = jnp.minimum(mv, edge_v[1, pl.ds(soff + b * _L, _L)])
        nz = jnp.sum(jnp.where(mv == 0, 1, 0))

        @pl.when(nz > 0)
        def _scan_fine():
            lax.fori_loop(s * _SUPER, (s + 1) * _SUPER, fine_step, 0)

        return carry

    lax.fori_loop(0, nsuper, super_step, 0)

    for r in range(n_rel):
        acc_v[_CNT_ROW, pl.ds(r * _L, _L)] = cnt_v[r, :]
    pltpu.sync_copy(acc_v, part_hbm.at[wid])


def _tc_tail_body(n_rel, part_ref, comp_ref, bases_ref, root_ref,
                  bias_ref, x0_ref, wg_ref, bg_ref, ws_ref, bs_ref,
                  outg_ref, outs_ref):
    s = jnp.sum(part_ref[...], axis=0)
    rows = []
    for r in range(n_rel):
        cnt_r = jnp.sum(s[_CNT_ROW, r * _L:(r + 1) * _L])
        rows.append(s[r:r + 1, :] / jnp.maximum(cnt_r, 1.0))
    m = jnp.concatenate(rows, axis=0)
    v = lax.dot_general(comp_ref[...], m,
                        (((0,), (0,)), ((), ())),
                        preferred_element_type=jnp.float32)
    out0 = jnp.dot(x0_ref[...], root_ref[...],
                   preferred_element_type=jnp.float32) + bias_ref[...]
    for r in range(n_rel):
        out0 = out0 + jnp.dot(v[r:r + 1, :], bases_ref[r],
                              preferred_element_type=jnp.float32)
    h = jnp.maximum(out0, 0.0)

    def head(w_ref, b_ref, o_ref):
        lg = lax.dot_general(h, w_ref[...], (((1,), (1,)), ((), ())),
                             preferred_element_type=jnp.float32) + b_ref[...]
        mx = jnp.max(lg, axis=1, keepdims=True)
        lse = jnp.log(jnp.sum(jnp.exp(lg - mx), axis=1, keepdims=True))
        o_ref[...] = lg - mx - lse

    head(wg_ref, bg_ref, outg_ref)
    head(ws_ref, bs_ref, outs_ref)


def kernel(batch_x, batch_edge_index, batch_edge_type, comp, bases, root,
           bias, W_global, b_global, W_sense, b_sense):
    n, d = batch_x.shape
    e = batch_edge_type.shape[0]
    n_rel = comp.shape[0]
    g = W_global.shape[0]
    s_cnt = W_sense.shape[0]

    info = plsc.get_sparse_core_info()
    nw = info.num_cores * info.num_subcores
    chunk = e // nw
    nblk = (chunk + _L - 1) // _L
    nsuper = (nblk + _SUPER - 1) // _SUPER
    win = ((chunk + 2 * _ALIGN + _ALIGN - 1) // _ALIGN) * _ALIGN
    buf_minor = (win - chunk) + nsuper * _SUPER * _L

    sc_fn = pl.kernel(
        functools.partial(_sc_scan_body, chunk, win, nsuper, n_rel,
                          info.num_cores, e),
        out_type=jax.ShapeDtypeStruct((nw, _ACC_ROWS, d), jnp.float32),
        mesh=plsc.VectorSubcoreMesh(core_axis_name="c", subcore_axis_name="s"),
        compiler_params=pltpu.CompilerParams(needs_layout_passes=False),
        scratch_types=[
            pltpu.VMEM((2, buf_minor), jnp.int32),
            pltpu.VMEM((nblk * _L,), jnp.int32),
            pltpu.VMEM((_L,), jnp.int32),
            pltpu.VMEM((_L, d), jnp.float32),
            pltpu.VMEM((_ACC_ROWS, d), jnp.float32),
            pltpu.VMEM((_ACC_ROWS, _L), jnp.float32),
            pltpu.SemaphoreType.DMA,
        ],
    )
    part = sc_fn(batch_x, batch_edge_index, batch_edge_type)

    x0 = lax.slice(batch_x, (0, 0), (1, d))
    outg, outs = pl.pallas_call(
        functools.partial(_tc_tail_body, n_rel),
        out_shape=(jax.ShapeDtypeStruct((1, g), jnp.float32),
                   jax.ShapeDtypeStruct((1, s_cnt), jnp.float32)),
    )(part, comp, bases, root, bias.reshape(1, d), x0,
      W_global, b_global.reshape(1, g), W_sense, b_sense.reshape(1, s_cnt))

    return (outg.reshape(g), outs.reshape(s_cnt))
